# baseline (speedup 1.0000x reference)
import functools

import numpy as np
import jax
import jax.numpy as jnp
from jax import lax
from jax.experimental import pallas as pl
from jax.experimental.pallas import tpu as pltpu
from jax.experimental.pallas import tpu_sc as plsc

_ROW_DEPTH = 30
_LANES = 16


def _perm_rc(n_seq: int, r_depth: int) -> np.ndarray:
    n = int(np.ceil(n_seq / r_depth) * r_depth)
    nb_rows = n // r_depth
    ind = np.arange(n, dtype=np.int32)
    perm_filler = ind.reshape(nb_rows, r_depth).T.reshape(-1)
    return perm_filler[perm_filler < n_seq].astype(np.int32)


@functools.cache
def _build(batch: int, n_seq: int):
    info = plsc.get_sparse_core_info()
    n_workers = info.num_cores * info.num_subcores
    assert batch % n_workers == 0
    assert n_seq % _LANES == 0
    rows_per_worker = batch // n_workers
    n_vec = n_seq // _LANES

    mesh = plsc.VectorSubcoreMesh(core_axis_name="c", subcore_axis_name="s")

    @functools.partial(
        pl.kernel,
        mesh=mesh,
        out_type=jax.ShapeDtypeStruct((batch, n_seq), jnp.float32),
        scratch_types=[
            pltpu.VMEM((n_seq,), jnp.int32),
            pltpu.VMEM((n_seq,), jnp.float32),
            pltpu.VMEM((n_seq,), jnp.float32),
        ],
        compiler_params=pltpu.CompilerParams(needs_layout_passes=False),
    )
    def interleave(in_hbm, perm_hbm, out_hbm, perm_v, in_v, out_v):
        wid = lax.axis_index("s") * info.num_cores + lax.axis_index("c")
        pltpu.sync_copy(perm_hbm, perm_v)

        def do_row(j, carry):
            row = wid * rows_per_worker + j
            pltpu.sync_copy(in_hbm.at[row], in_v)

            def gather16(i, c):
                idx = perm_v[pl.ds(i * _LANES, _LANES)]
                out_v[pl.ds(i * _LANES, _LANES)] = plsc.load_gather(
                    in_v, [idx]
                )
                return c

            lax.fori_loop(0, n_vec, gather16, 0, unroll=4)
            pltpu.sync_copy(out_v, out_hbm.at[row])
            return carry

        lax.fori_loop(0, rows_per_worker, do_row, 0)

    return interleave


def kernel(inputs):
    batch, n_seq = inputs.shape
    perm = jnp.asarray(_perm_rc(n_seq, _ROW_DEPTH))
    return _build(batch, n_seq)(inputs, perm)


# speedup vs baseline: 2.2424x; 2.2424x over previous
import functools

import numpy as np
import jax
import jax.numpy as jnp
from jax import lax
from jax.experimental import pallas as pl
from jax.experimental.pallas import tpu as pltpu
from jax.experimental.pallas import tpu_sc as plsc

_ROW_DEPTH = 30
_LANES = 16


def _perm_rc(n_seq: int, r_depth: int) -> np.ndarray:
    n = int(np.ceil(n_seq / r_depth) * r_depth)
    nb_rows = n // r_depth
    ind = np.arange(n, dtype=np.int32)
    perm_filler = ind.reshape(nb_rows, r_depth).T.reshape(-1)
    return perm_filler[perm_filler < n_seq].astype(np.int32)


@functools.cache
def _build(batch: int, n_seq: int):
    info = plsc.get_sparse_core_info()
    n_workers = info.num_cores * info.num_subcores
    assert batch % n_workers == 0
    assert n_seq % _LANES == 0
    rows_per_worker = batch // n_workers
    n_vec = n_seq // _LANES

    mesh = plsc.VectorSubcoreMesh(core_axis_name="c", subcore_axis_name="s")

    @functools.partial(
        pl.kernel,
        mesh=mesh,
        out_type=jax.ShapeDtypeStruct((batch, n_seq), jnp.float32),
        scratch_types=[
            pltpu.VMEM((n_seq,), jnp.int32),
            pltpu.VMEM((n_seq,), jnp.float32),
            pltpu.VMEM((n_seq,), jnp.float32),
        ],
        compiler_params=pltpu.CompilerParams(needs_layout_passes=False),
    )
    def interleave(in_hbm, perm_hbm, out_hbm, perm_v, in_v, out_v):
        wid = lax.axis_index("s") * info.num_cores + lax.axis_index("c")
        pltpu.sync_copy(perm_hbm, perm_v)

        def do_row(j, carry):
            row = wid * rows_per_worker + j
            pltpu.sync_copy(in_hbm.at[row], in_v)

            @plsc.parallel_loop(0, n_seq, step=_LANES, unroll=8)
            def gather16(i):
                idx = perm_v[pl.ds(i, _LANES)]
                out_v[pl.ds(i, _LANES)] = plsc.load_gather(
                    in_v, [idx]
                )
            pltpu.sync_copy(out_v, out_hbm.at[row])
            return carry

        lax.fori_loop(0, rows_per_worker, do_row, 0)

    return interleave


def kernel(inputs):
    batch, n_seq = inputs.shape
    perm = jnp.asarray(_perm_rc(n_seq, _ROW_DEPTH))
    return _build(batch, n_seq)(inputs, perm)


# speedup vs baseline: 2.7967x; 1.2472x over previous
import functools

import numpy as np
import jax
import jax.numpy as jnp
from jax import lax
from jax.experimental import pallas as pl
from jax.experimental.pallas import tpu as pltpu
from jax.experimental.pallas import tpu_sc as plsc

_ROW_DEPTH = 30
_LANES = 16
_OUT_CHUNKS = 4


def _perm_rc(n_seq: int, r_depth: int) -> np.ndarray:
    n = int(np.ceil(n_seq / r_depth) * r_depth)
    nb_rows = n // r_depth
    ind = np.arange(n, dtype=np.int32)
    perm_filler = ind.reshape(nb_rows, r_depth).T.reshape(-1)
    return perm_filler[perm_filler < n_seq].astype(np.int32)


@functools.cache
def _build(batch: int, n_seq: int):
    info = plsc.get_sparse_core_info()
    n_workers = info.num_cores * info.num_subcores
    assert batch % n_workers == 0
    assert n_seq % (_LANES * _OUT_CHUNKS) == 0
    rows_per_worker = batch // n_workers
    chunk = n_seq // _OUT_CHUNKS

    mesh = plsc.VectorSubcoreMesh(core_axis_name="c", subcore_axis_name="s")

    @functools.partial(
        pl.kernel,
        mesh=mesh,
        out_type=jax.ShapeDtypeStruct((batch, n_seq), jnp.float32),
        scratch_types=[
            pltpu.VMEM((n_seq,), jnp.int32),
            pltpu.VMEM((n_seq,), jnp.float32),
            pltpu.VMEM((n_seq,), jnp.float32),
            pltpu.VMEM((chunk,), jnp.float32),
            pltpu.VMEM((chunk,), jnp.float32),
            pltpu.SemaphoreType.DMA,
            pltpu.SemaphoreType.DMA,
            pltpu.SemaphoreType.DMA,
            pltpu.SemaphoreType.DMA,
        ],
        compiler_params=pltpu.CompilerParams(needs_layout_passes=False),
    )
    def interleave(in_hbm, perm_hbm, out_hbm, perm_v, in_v0, in_v1,
                   out_v0, out_v1, sem_i0, sem_i1, sem_o0, sem_o1):
        wid = lax.axis_index("s") * info.num_cores + lax.axis_index("c")
        row0 = wid * rows_per_worker
        in_bufs, sem_ins = [in_v0, in_v1], [sem_i0, sem_i1]
        out_bufs, sem_outs = [out_v0, out_v1], [sem_o0, sem_o1]

        h_in = [None, None]
        h_out = [None, None]
        h_in[0] = pltpu.async_copy(in_hbm.at[row0], in_bufs[0], sem_ins[0])
        pltpu.sync_copy(perm_hbm, perm_v)

        for j in range(rows_per_worker):
            jb = j % 2
            h_in[jb].wait()
            if j + 1 < rows_per_worker:
                nb = (j + 1) % 2
                h_in[nb] = pltpu.async_copy(
                    in_hbm.at[row0 + j + 1], in_bufs[nb], sem_ins[nb]
                )
            for k in range(_OUT_CHUNKS):
                b = (j * _OUT_CHUNKS + k) % 2
                if h_out[b] is not None:
                    h_out[b].wait()
                src, dst = in_bufs[jb], out_bufs[b]

                @plsc.parallel_loop(0, chunk, step=_LANES, unroll=8)
                def gather16(i, _base=k * chunk, _src=src, _dst=dst):
                    idx = perm_v[pl.ds(_base + i, _LANES)]
                    _dst[pl.ds(i, _LANES)] = plsc.load_gather(_src, [idx])

                h_out[b] = pltpu.async_copy(
                    dst,
                    out_hbm.at[row0 + j, pl.ds(k * chunk, chunk)],
                    sem_outs[b],
                )
        h_out[0].wait()
        h_out[1].wait()

    return interleave


def kernel(inputs):
    batch, n_seq = inputs.shape
    perm = jnp.asarray(_perm_rc(n_seq, _ROW_DEPTH))
    return _build(batch, n_seq)(inputs, perm)
